# baseline (speedup 1.0000x reference)
import functools

import jax
import jax.numpy as jnp
from jax import lax
from jax.experimental import pallas as pl
from jax.experimental.pallas import tpu as pltpu
from jax.experimental.pallas import tpu_sc as plsc

_M, _H = 8192, 1024
_NC, _NS = 2, 16
_NW = _NC * _NS
_RPW = _M // _NW
_CH = 16
_NBUF = 7
_K = 3
_NCHUNK = _RPW // _CH


def _sc_copy_body(table, out, buf, isem, osem):
    c = lax.axis_index("c")
    s = lax.axis_index("s")
    wid = s * _NC + c
    base = wid * _RPW
    loads = [
        pltpu.make_async_copy(
            table.at[pl.ds(base + i * _CH, _CH)], buf.at[i % _NBUF],
            isem.at[i % _NBUF],
        )
        for i in range(_NCHUNK)
    ]
    stores = [
        pltpu.make_async_copy(
            buf.at[i % _NBUF], out.at[pl.ds(base + i * _CH, _CH)],
            osem.at[i % _NBUF],
        )
        for i in range(_NCHUNK)
    ]
    for i in range(min(_K, _NCHUNK)):
        loads[i].start()
    for i in range(_NCHUNK):
        loads[i].wait()
        stores[i].start()
        j = i + _K
        if j < _NCHUNK:
            if j - _NBUF >= 0:
                stores[j - _NBUF].wait()
            loads[j].start()
    for i in range(max(0, _NCHUNK - _NBUF), _NCHUNK):
        stores[i].wait()


@functools.partial(
    pl.kernel,
    mesh=plsc.VectorSubcoreMesh(core_axis_name="c", subcore_axis_name="s"),
    out_type=jax.ShapeDtypeStruct((_M, _H), jnp.float32),
    scratch_types=[
        pltpu.VMEM((_NBUF, _CH, _H), jnp.float32),
        pltpu.SemaphoreType.DMA((_NBUF,)),
        pltpu.SemaphoreType.DMA((_NBUF,)),
    ],
)
def _sc_copy(table, out, buf, isem, osem):
    _sc_copy_body(table, out, buf, isem, osem)


def kernel(seq_len, position_embedding):
    del seq_len
    return _sc_copy(position_embedding)


# speedup vs baseline: 2.0179x; 2.0179x over previous
import jax
import jax.numpy as jnp
from jax.experimental import pallas as pl
from jax.experimental.pallas import tpu as pltpu

_BOUNDS = (0, 4096, 8192)


def _dma_copy_kernel(in_ref, out_ref, vbuf, in_sem, out_sem):
    n = len(_BOUNDS) - 1
    loads = [
        pltpu.make_async_copy(
            in_ref.at[pl.ds(_BOUNDS[k], _BOUNDS[k + 1] - _BOUNDS[k])],
            vbuf.at[pl.ds(_BOUNDS[k], _BOUNDS[k + 1] - _BOUNDS[k])],
            in_sem.at[k],
        )
        for k in range(n)
    ]
    stores = [
        pltpu.make_async_copy(
            vbuf.at[pl.ds(_BOUNDS[k], _BOUNDS[k + 1] - _BOUNDS[k])],
            out_ref.at[pl.ds(_BOUNDS[k], _BOUNDS[k + 1] - _BOUNDS[k])],
            out_sem.at[k],
        )
        for k in range(n)
    ]
    for k in range(n):
        loads[k].start()
    for k in range(n):
        loads[k].wait()
        stores[k].start()
    for k in range(n):
        stores[k].wait()


def kernel(seq_len, position_embedding):
    del seq_len
    M, H = position_embedding.shape
    n = len(_BOUNDS) - 1
    return pl.pallas_call(
        _dma_copy_kernel,
        in_specs=[pl.BlockSpec(memory_space=pltpu.MemorySpace.HBM)],
        out_specs=pl.BlockSpec(memory_space=pltpu.MemorySpace.HBM),
        out_shape=jax.ShapeDtypeStruct((M, H), position_embedding.dtype),
        scratch_shapes=[
            pltpu.VMEM((M, H), position_embedding.dtype),
            pltpu.SemaphoreType.DMA((n,)),
            pltpu.SemaphoreType.DMA((n,)),
        ],
    )(position_embedding)
